# Initial kernel scaffold; baseline (speedup 1.0000x reference)
import jax
import jax.numpy as jnp
from jax.experimental import pallas as pl


def kernel(x, edge_index, gcn_W, gcn_b, W_ih, W_hh, b_ih, b_hh, lin_W, lin_b):
    raise NotImplementedError("write your pallas kernel here")


# speedup vs baseline: 16.0943x; 16.0943x over previous
import functools

import jax
import jax.numpy as jnp
from jax import lax
from jax.experimental import pallas as pl
from jax.experimental.pallas import tpu as pltpu
from jax.experimental.pallas import tpu_sc as plsc

S = 4
N = 50000
E = 3200000

NC = 2
NS = 16
NW = NC * NS


_EW_A = E // NW
_WIN_A = 10000
_NWIN_A = _EW_A // _WIN_A


def _deg_body(dst_hbm, deg_hbm, hist, ebuf):
    wid = lax.axis_index("s") * NC + lax.axis_index("c")
    ones = jnp.full((16,), 1.0, jnp.float32)
    for g in range(S):
        @plsc.parallel_loop(0, N // 16, step=1)
        def _zero(i):
            hist[pl.ds(i * 16, 16)] = jnp.zeros((16,), jnp.float32)

        base = g * E + wid * _EW_A
        for w in range(_NWIN_A):
            pltpu.sync_copy(dst_hbm.at[pl.ds(base + w * _WIN_A, _WIN_A)],
                            ebuf)

            def _scat(i, carry):
                idx = ebuf[pl.ds(i * 16, 16)]
                plsc.addupdate_scatter(hist, [idx], ones)
                return carry

            lax.fori_loop(0, _WIN_A // 16, _scat, 0)

        pltpu.sync_copy(hist, deg_hbm.at[pl.ds((g * NW + wid) * N, N)])


def _deg_partials(dst_flat):
    mesh = plsc.VectorSubcoreMesh(core_axis_name="c", subcore_axis_name="s")
    return pl.kernel(
        _deg_body,
        out_type=jax.ShapeDtypeStruct((S * NW * N,), jnp.float32),
        mesh=mesh,
        compiler_params=pltpu.CompilerParams(needs_layout_passes=False),
        scratch_types=[
            pltpu.VMEM((N,), jnp.float32),
            pltpu.VMEM((_WIN_A,), jnp.int32),
        ],
    )(dst_flat)


def _b_body(deg_ref, xt_ref, w_ref, dinv_ref, u0_ref, u1_ref):
    deg = jnp.sum(deg_ref[0], axis=0) + 1.0
    dinv = lax.rsqrt(deg)
    x0 = xt_ref[0, 0]
    x1 = xt_ref[0, 1]
    w00 = w_ref[0, 0]
    w01 = w_ref[0, 1]
    w10 = w_ref[1, 0]
    w11 = w_ref[1, 1]
    dinv_ref[0, 0] = dinv
    u0_ref[0, 0] = (x0 * w00 + x1 * w10) * dinv
    u1_ref[0, 0] = (x0 * w01 + x1 * w11) * dinv


def _dinv_u(deg_part, xt, gcn_W):
    return pl.pallas_call(
        _b_body,
        grid=(S,),
        in_specs=[
            pl.BlockSpec((1, NW, N), lambda g: (g, 0, 0)),
            pl.BlockSpec((1, 2, N), lambda g: (g, 0, 0)),
            pl.BlockSpec(memory_space=pltpu.SMEM),
        ],
        out_specs=[
            pl.BlockSpec((1, 1, N), lambda g: (g, 0, 0)),
            pl.BlockSpec((1, 1, N), lambda g: (g, 0, 0)),
            pl.BlockSpec((1, 1, N), lambda g: (g, 0, 0)),
        ],
        out_shape=[jax.ShapeDtypeStruct((S, 1, N), jnp.float32)] * 3,
    )(deg_part, xt, gcn_W)


_EW_C = E // (NW // 2)
_WIN_C = 8000
_NWIN_C = _EW_C // _WIN_C


def _scat_body(src_hbm, dst_hbm, u0_hbm, u1_hbm, acc_hbm,
               table, acc, sbuf, dbuf):
    wid = lax.axis_index("s") * NC + lax.axis_index("c")
    col = wid % 2
    kk = wid // 2
    for g in range(S):
        @pl.when(col == 0)
        def _():
            pltpu.sync_copy(u0_hbm.at[pl.ds(g * N, N)], table)

        @pl.when(col == 1)
        def _():
            pltpu.sync_copy(u1_hbm.at[pl.ds(g * N, N)], table)

        @plsc.parallel_loop(0, N // 16, step=1)
        def _zero(i):
            acc[pl.ds(i * 16, 16)] = jnp.zeros((16,), jnp.float32)

        base = g * E + kk * _EW_C
        for w in range(_NWIN_C):
            off = base + w * _WIN_C
            pltpu.sync_copy(src_hbm.at[pl.ds(off, _WIN_C)], sbuf)
            pltpu.sync_copy(dst_hbm.at[pl.ds(off, _WIN_C)], dbuf)

            def _edge(i, carry):
                sidx = sbuf[pl.ds(i * 16, 16)]
                m = plsc.load_gather(table, [sidx])
                didx = dbuf[pl.ds(i * 16, 16)]
                plsc.addupdate_scatter(acc, [didx], m)
                return carry

            lax.fori_loop(0, _WIN_C // 16, _edge, 0)

        pltpu.sync_copy(acc,
                        acc_hbm.at[pl.ds(((g * 2 + col) * (NW // 2) + kk) * N,
                                         N)])


def _scatter_partials(src_flat, dst_flat, u0_flat, u1_flat):
    mesh = plsc.VectorSubcoreMesh(core_axis_name="c", subcore_axis_name="s")
    return pl.kernel(
        _scat_body,
        out_type=jax.ShapeDtypeStruct((S * 2 * (NW // 2) * N,), jnp.float32),
        mesh=mesh,
        compiler_params=pltpu.CompilerParams(needs_layout_passes=False),
        scratch_types=[
            pltpu.VMEM((N,), jnp.float32),
            pltpu.VMEM((N,), jnp.float32),
            pltpu.VMEM((_WIN_C,), jnp.int32),
            pltpu.VMEM((_WIN_C,), jnp.int32),
        ],
    )(src_flat, dst_flat, u0_flat, u1_flat)


def _d1_body(acc_ref, u0_ref, u1_ref, dinv_ref, gb_ref, wih_ref, bsum_ref,
             a_ref):
    dinv = dinv_ref[0, 0]
    acc0 = jnp.sum(acc_ref[0, 0], axis=0)
    acc1 = jnp.sum(acc_ref[0, 1], axis=0)
    out0 = dinv * (acc0 + u0_ref[0, 0]) + gb_ref[0]
    out1 = dinv * (acc1 + u1_ref[0, 0]) + gb_ref[1]
    for j in range(3):
        a_ref[j, 0, 0] = (out0 * wih_ref[j, 0] + out1 * wih_ref[j, 1]
                          + bsum_ref[j])


def _rnn_inputs(acc_part, u0, u1, dinv, gcn_b, W_ih, bsum):
    return pl.pallas_call(
        _d1_body,
        grid=(S,),
        in_specs=[
            pl.BlockSpec((1, 2, NW // 2, N), lambda g: (g, 0, 0, 0)),
            pl.BlockSpec((1, 1, N), lambda g: (g, 0, 0)),
            pl.BlockSpec((1, 1, N), lambda g: (g, 0, 0)),
            pl.BlockSpec((1, 1, N), lambda g: (g, 0, 0)),
            pl.BlockSpec(memory_space=pltpu.SMEM),
            pl.BlockSpec(memory_space=pltpu.SMEM),
            pl.BlockSpec(memory_space=pltpu.SMEM),
        ],
        out_specs=pl.BlockSpec((3, 1, 1, N), lambda g: (0, g, 0, 0)),
        out_shape=jax.ShapeDtypeStruct((3, S, 1, N), jnp.float32),
    )(acc_part, u0, u1, dinv, gcn_b, W_ih, bsum)


_T = S * N
_PAD = 192
_TP = _T + _PAD
_ROWS = _TP * 4 // 1024


def _d2_body(p_ref, c_ref, lmask_ref, lb_ref, out_ref):
    cm2 = c_ref[0, :]
    cm1 = c_ref[1, :]
    c0 = c_ref[2, :]
    cp1 = c_ref[3, :]
    cp2 = c_ref[4, :]

    def row_step(r, h):
        tile = p_ref[r]
        for sub in range(8):
            arow = tile[sub, :]
            for s in range(32):
                a = arow if s == 0 else jnp.roll(arow, -4 * s)
                z = (a + c0 * h
                     + cm1 * jnp.roll(h, -1) + cm2 * jnp.roll(h, -2)
                     + cp1 * jnp.roll(h, 1) + cp2 * jnp.roll(h, 2))
                h = jnp.tanh(z)
        return h

    h = jnp.zeros((128,), jnp.float32)
    h = lax.fori_loop(0, _ROWS, row_step, h)
    z = jnp.sum(jnp.maximum(h, 0.0) * lmask_ref[0, :]) + lb_ref[0, 0]
    zv = jnp.full((128,), z, jnp.float32)
    sig = 1.0 / (1.0 + jnp.exp(-zv))
    out_ref[...] = sig[:1].reshape(1, 1)


def _rnn_tail(p, cmat, lmask, lin_b):
    return pl.pallas_call(
        _d2_body,
        in_specs=[
            pl.BlockSpec(memory_space=pltpu.VMEM),
            pl.BlockSpec(memory_space=pltpu.VMEM),
            pl.BlockSpec(memory_space=pltpu.VMEM),
            pl.BlockSpec(memory_space=pltpu.SMEM),
        ],
        out_specs=pl.BlockSpec(memory_space=pltpu.VMEM),
        out_shape=jax.ShapeDtypeStruct((1, 1), jnp.float32),
    )(p, cmat, lmask, lin_b)


def kernel(x, edge_index, gcn_W, gcn_b, W_ih, W_hh, b_ih, b_hh, lin_W, lin_b):
    src = edge_index[:, 0, :].reshape(S * E)
    dst = edge_index[:, 1, :].reshape(S * E)
    xt = x.transpose(0, 2, 1)

    deg_part = _deg_partials(dst).reshape(S, NW, N)
    dinv, u0, u1 = _dinv_u(deg_part, xt, gcn_W)
    acc_part = _scatter_partials(src, dst, u0.reshape(S * N),
                                 u1.reshape(S * N)).reshape(S, 2, NW // 2, N)
    bsum = b_ih + b_hh
    a = _rnn_inputs(acc_part, u0, u1, dinv, gcn_b, W_ih, bsum)

    af = a.reshape(3, _T)
    p = jnp.stack([af[0], af[1], af[2], jnp.zeros((_T,), jnp.float32)],
                  axis=1)
    p = jnp.concatenate([jnp.zeros((_PAD, 4), jnp.float32), p], axis=0)
    p = p.reshape(_ROWS, 8, 128)

    jj = jnp.arange(128)
    cmat = jnp.stack([
        jnp.where((jj < 3) & (jj - d >= 0) & (jj - d < 3),
                  W_hh[jnp.clip(jj, 0, 2), jnp.clip(jj - d, 0, 2)],
                  0.0)
        for d in (-2, -1, 0, 1, 2)
    ], axis=0).astype(jnp.float32)

    lmask = jnp.zeros((1, 128), jnp.float32).at[0, :3].set(lin_W[0, :])

    return _rnn_tail(p, cmat, lmask, lin_b.reshape(1, 1))
